# Initial kernel scaffold; baseline (speedup 1.0000x reference)
import jax
import jax.numpy as jnp
from jax.experimental import pallas as pl


def kernel(x, edge_index, batch, W1, a1s, a1d, b1, Wp1, bp1, W2, a2s, a2d, b2, Wp2, bp2, W3, a3s, a3d, b3, Wl, bl):
    raise NotImplementedError("write your pallas kernel here")


# speedup vs baseline: 17.5193x; 17.5193x over previous
import functools

import jax
import jax.numpy as jnp
from jax import lax
from jax.experimental import pallas as pl
from jax.experimental.pallas import tpu as pltpu
from jax.experimental.pallas import tpu_sc as plsc

N = 10000
E = 160000
HI = jax.lax.Precision.HIGHEST
F32 = jnp.float32

NC, NS, L = 2, 16, 16
CPR = 624
TAIL0 = NS * CPR
TAILN = N - TAIL0
EPT = E // NS
CHUNK = 80
NCHUNK = EPT // CHUNK
DLOC = 4 * N
DPAD = 40960

_MESH = plsc.VectorSubcoreMesh(
    core_axis_name="c", subcore_axis_name="s", num_cores=NC, num_subcores=NS)


def _iota16():
    return lax.iota(jnp.int32, L)


def _splat_i32(val):
    return jnp.zeros((L,), jnp.int32) + val


def _gat_sweep(src_ref, dst_ref, hsp_ref, aval_ref, zero_ref, zden_ref,
               out_ref, den_ref,
               sidx, didx, hidx, ia0, ia1, ia2, ia3, va0, va1, va2, va3,
               hbuf, comb, denpay, acc_sh, den_sh, sem):
    c = lax.axis_index("c")
    s = lax.axis_index("s")
    r0 = s * CPR
    pltpu.sync_copy(zero_ref.at[pl.ds(r0, CPR)], acc_sh.at[pl.ds(r0, CPR)])
    pltpu.sync_copy(zden_ref.at[pl.ds(r0, CPR)], den_sh.at[pl.ds(r0, CPR)])

    @pl.when(s == 0)
    def _():
        pltpu.sync_copy(zero_ref.at[pl.ds(TAIL0, TAILN)],
                        acc_sh.at[pl.ds(TAIL0, TAILN)])
        pltpu.sync_copy(zden_ref.at[pl.ds(TAIL0, TAILN)],
                        den_sh.at[pl.ds(TAIL0, TAILN)])

    plsc.subcore_barrier()
    ones = jnp.zeros((L,), F32) + 1.0
    zeros = jnp.zeros((L,), F32)

    def body(k, _):
        base = s * EPT + k * CHUNK
        pltpu.sync_copy(src_ref.at[pl.ds(base, CHUNK)], sidx)
        pltpu.sync_copy(dst_ref.at[pl.ds(base, CHUNK)], didx)
        for g in range(CHUNK // L):
            sv = sidx[pl.ds(g * L, L)]
            dv = didx[pl.ds(g * L, L)]
            hidx[pl.ds(g * L, L)] = sv * 2 + c
            s8 = sv * 8 + 2 * c
            d8 = dv * 8 + 4 + 2 * c
            ia0[pl.ds(g * L, L)] = s8
            ia1[pl.ds(g * L, L)] = s8 + 1
            ia2[pl.ds(g * L, L)] = d8
            ia3[pl.ds(g * L, L)] = d8 + 1
        pltpu.async_copy(aval_ref.at[ia0], va0, sem).wait()
        pltpu.async_copy(aval_ref.at[ia1], va1, sem).wait()
        pltpu.async_copy(aval_ref.at[ia2], va2, sem).wait()
        pltpu.async_copy(aval_ref.at[ia3], va3, sem).wait()
        pltpu.async_copy(hsp_ref.at[hidx], hbuf, sem).wait()
        for g in range(CHUNK // L):
            sl = pl.ds(g * L, L)
            rows = _iota16() + g * L
            for hh, (va, vd) in enumerate(((va0, va2), (va1, va3))):
                lg = va[sl] + vd[sl]
                ex = jnp.exp(jnp.maximum(lg, 0.2 * lg))
                va[sl] = ex
                plsc.store_scatter(denpay, [rows, _splat_i32(hh)], ex)
            plsc.store_scatter(denpay, [rows, _splat_i32(2)], ones)
            plsc.store_scatter(denpay, [rows, _splat_i32(3)], zeros)
        for e in range(CHUNK):
            erow = _splat_i32(e)
            sp0 = plsc.load_gather(va0, [erow])
            sp1 = plsc.load_gather(va1, [erow])
            for q in range(4):
                comb[e, pl.ds(q * L, L)] = hbuf[e, pl.ds(q * L, L)] * sp0
            for q in range(4, 8):
                comb[e, pl.ds(q * L, L)] = hbuf[e, pl.ds(q * L, L)] * sp1
        pltpu.sync_copy(comb, acc_sh.at[didx], add=True)
        pltpu.sync_copy(denpay, den_sh.at[didx], add=True)
        return ()

    lax.fori_loop(0, NCHUNK, body, ())
    plsc.subcore_barrier()
    pltpu.sync_copy(acc_sh.at[pl.ds(r0, CPR)],
                    out_ref.at[c].at[pl.ds(r0, CPR)])
    pltpu.sync_copy(den_sh.at[pl.ds(r0, CPR)],
                    den_ref.at[c].at[pl.ds(r0, CPR)])

    @pl.when(s == 0)
    def _():
        pltpu.sync_copy(acc_sh.at[pl.ds(TAIL0, TAILN)],
                        out_ref.at[c].at[pl.ds(TAIL0, TAILN)])
        pltpu.sync_copy(den_sh.at[pl.ds(TAIL0, TAILN)],
                        den_ref.at[c].at[pl.ds(TAIL0, TAILN)])


_gat_sweep_call = functools.partial(
    pl.kernel,
    out_type=(jax.ShapeDtypeStruct((NC, N, 128), F32),
              jax.ShapeDtypeStruct((NC, N, 4), F32)),
    mesh=_MESH,
    scratch_types=[
        pltpu.VMEM((CHUNK,), jnp.int32),
        pltpu.VMEM((CHUNK,), jnp.int32),
        pltpu.VMEM((CHUNK,), jnp.int32),
        pltpu.VMEM((CHUNK,), jnp.int32),
        pltpu.VMEM((CHUNK,), jnp.int32),
        pltpu.VMEM((CHUNK,), jnp.int32),
        pltpu.VMEM((CHUNK,), jnp.int32),
        pltpu.VMEM((CHUNK,), F32),
        pltpu.VMEM((CHUNK,), F32),
        pltpu.VMEM((CHUNK,), F32),
        pltpu.VMEM((CHUNK,), F32),
        pltpu.VMEM((CHUNK, 128), F32),
        pltpu.VMEM((CHUNK, 128), F32),
        pltpu.VMEM((CHUNK, 4), F32),
        pltpu.VMEM_SHARED((N, 128), F32),
        pltpu.VMEM_SHARED((N, 4), F32),
        pltpu.SemaphoreType.DMA,
    ],
    compiler_params=pltpu.CompilerParams(use_tc_tiling_on_sc=False,
                                        needs_layout_passes=False),
)(_gat_sweep)


def _seg_sweep(src_ref, dst_ref, tab_ref, zero_ref, out_ref,
               sidx, didx, hidx, buf, acc_sh, sem):
    c = lax.axis_index("c")
    s = lax.axis_index("s")
    r0 = s * CPR
    pltpu.sync_copy(zero_ref.at[pl.ds(r0, CPR)], acc_sh.at[pl.ds(r0, CPR)])

    @pl.when(s == 0)
    def _():
        pltpu.sync_copy(zero_ref.at[pl.ds(TAIL0, TAILN)],
                        acc_sh.at[pl.ds(TAIL0, TAILN)])

    plsc.subcore_barrier()

    def body(k, _):
        base = s * EPT + k * CHUNK
        pltpu.sync_copy(src_ref.at[pl.ds(base, CHUNK)], sidx)
        pltpu.sync_copy(dst_ref.at[pl.ds(base, CHUNK)], didx)
        for g in range(CHUNK // L):
            v = sidx[pl.ds(g * L, L)]
            hidx[pl.ds(g * L, L)] = v * 2 + c
        pltpu.async_copy(tab_ref.at[hidx], buf, sem).wait()
        pltpu.sync_copy(buf, acc_sh.at[didx], add=True)
        return ()

    lax.fori_loop(0, NCHUNK, body, (), unroll=False)
    plsc.subcore_barrier()
    pltpu.sync_copy(acc_sh.at[pl.ds(r0, CPR)],
                    out_ref.at[c].at[pl.ds(r0, CPR)])

    @pl.when(s == 0)
    def _():
        pltpu.sync_copy(acc_sh.at[pl.ds(TAIL0, TAILN)],
                        out_ref.at[c].at[pl.ds(TAIL0, TAILN)])


_seg_sweep_call = functools.partial(
    pl.kernel,
    out_type=jax.ShapeDtypeStruct((NC, N, 16), F32),
    mesh=_MESH,
    scratch_types=[
        pltpu.VMEM((CHUNK,), jnp.int32),
        pltpu.VMEM((CHUNK,), jnp.int32),
        pltpu.VMEM((CHUNK,), jnp.int32),
        pltpu.VMEM((CHUNK, 16), F32),
        pltpu.VMEM_SHARED((N, 16), F32),
        pltpu.SemaphoreType.DMA,
    ],
    compiler_params=pltpu.CompilerParams(use_tc_tiling_on_sc=False,
                                        needs_layout_passes=False),
)(_seg_sweep)


BLK = 1000
GRID = N // BLK


def _k1_body(x_ref, w1_ref, ab_ref, h_ref, aval_ref):
    h = jnp.dot(x_ref[...], w1_ref[...], precision=HI)
    h_ref[...] = h
    aval_ref[...] = jnp.dot(h, ab_ref[...], precision=HI)


def _k2b_body(dp_ref, den_ref):
    den_ref[...] = jnp.sum(dp_ref[...], axis=1)


def _k3_body(acc_ref, den_ref, b1_ref, wp1_ref, x1_ref, g_ref, dinv_ref):
    a0 = acc_ref[0]
    a1 = acc_ref[1]
    d0 = den_ref[0]
    d1 = den_ref[1]
    eps = 1e-16
    x1 = 0.25 * (a0[:, 0:64] / (d0[:, 0:1] + eps)
                 + a0[:, 64:128] / (d0[:, 1:2] + eps)
                 + a1[:, 0:64] / (d1[:, 0:1] + eps)
                 + a1[:, 64:128] / (d1[:, 1:2] + eps)) + b1_ref[...]
    deg = d0[:, 2:3] + 1.0
    dinv = lax.rsqrt(deg)
    h1 = jnp.dot(x1, wp1_ref[...], precision=HI)
    x1_ref[...] = x1
    g_ref[...] = dinv * h1
    dinv_ref[...] = jnp.broadcast_to(dinv, (BLK, 8))


def _k5_body(agg_ref, g_ref, dinv_ref, x1_ref, bp1_ref,
             ssoft_ref, xp_ref, xp_acc):
    i = pl.program_id(0)
    aggcat = jnp.concatenate([agg_ref[0], agg_ref[1]], axis=1)
    dinv = dinv_ref[:, 0:1]
    s1 = dinv * (aggcat + g_ref[...]) + bp1_ref[...]
    m = jnp.max(s1, axis=1, keepdims=True)
    ee = jnp.exp(s1 - m)
    ssoft = ee / jnp.sum(ee, axis=1, keepdims=True)
    ssoft_ref[...] = ssoft

    @pl.when(i == 0)
    def _():
        xp_acc[...] = jnp.zeros_like(xp_acc)

    xp_acc[...] += lax.dot_general(ssoft, x1_ref[...],
                                   (((0,), (0,)), ((), ())), precision=HI)

    @pl.when(i == GRID - 1)
    def _():
        xp_ref[...] = xp_acc[...]


def _gat_dense_tc(x, adj, W, a_s, a_d, b, H, D):
    h = jnp.dot(x, W, precision=HI)
    ladj = jnp.log(adj + 1e-9)
    acc = None
    for hh in range(H):
        hv = h[:, hh * D:(hh + 1) * D]
        als_col = lax.dot_general(hv, a_s[hh:hh + 1, :],
                                  (((1,), (1,)), ((), ())), precision=HI)
        ald_row = lax.dot_general(a_d[hh:hh + 1, :], hv,
                                  (((1,), (1,)), ((), ())), precision=HI)
        lg = als_col + ald_row
        Lm = jnp.maximum(lg, 0.2 * lg) + ladj
        m = jnp.max(Lm, axis=1, keepdims=True)
        ee = jnp.exp(Lm - m)
        alpha = ee / jnp.sum(ee, axis=1, keepdims=True)
        outh = jnp.dot(alpha, hv, precision=HI)
        acc = outh if acc is None else acc + outh
    return acc / H + b


def _k7_body(t_ref, ssoft_ref, xp_ref, w2_ref, a2s_ref, a2d_ref, b2_ref,
             wp2_ref, bp2_ref, w3_ref, a3s_ref, a3d_ref, b3_ref,
             wl_ref, bl_ref, out_ref, adj_acc):
    i = pl.program_id(0)

    @pl.when(i == 0)
    def _():
        adj_acc[...] = jnp.zeros_like(adj_acc)

    tcat = jnp.concatenate([t_ref[0], t_ref[1]], axis=1)
    adj_acc[...] += lax.dot_general(tcat, ssoft_ref[...],
                                    (((0,), (0,)), ((), ())), precision=HI)

    @pl.when(i == GRID - 1)
    def _():
        adjp = adj_acc[...]
        x2 = xp_ref[...]
        x3 = _gat_dense_tc(x2, adjp, w2_ref[...], a2s_ref[...], a2d_ref[...],
                           b2_ref[...], 4, 32)
        n = 32
        ii = lax.broadcasted_iota(jnp.int32, (n, n), 0)
        jj = lax.broadcasted_iota(jnp.int32, (n, n), 1)
        A = adjp + jnp.where(ii == jj, 1.0, 0.0)
        deg = jnp.sum(A, axis=1, keepdims=True)
        dinv = lax.rsqrt(deg)
        y = dinv * jnp.dot(x3, wp2_ref[...], precision=HI)
        s2 = dinv * jnp.dot(A, y, precision=HI) + bp2_ref[...]
        m = jnp.max(s2, axis=1, keepdims=True)
        ee = jnp.exp(s2 - m)
        s = ee / jnp.sum(ee, axis=1, keepdims=True)
        x4 = lax.dot_general(s, x3, (((0,), (0,)), ((), ())), precision=HI)
        adj3 = lax.dot_general(s, jnp.dot(adjp, s, precision=HI),
                               (((0,), (0,)), ((), ())), precision=HI)
        x5 = _gat_dense_tc(x4, adj3, w3_ref[...], a3s_ref[...], a3d_ref[...],
                           b3_ref[...], 4, 16)
        x6 = jnp.dot(x5, wl_ref[...], precision=HI) + bl_ref[...]
        out_ref[...] = jnp.mean(x6, axis=0, keepdims=True)


def _full(shape):
    return pl.BlockSpec(shape, lambda i: tuple(0 for _ in shape))


def _rows(shape):
    return pl.BlockSpec(shape, lambda i: (i,) + tuple(0 for _ in shape[1:]))


def _rows3(shape):
    return pl.BlockSpec(shape, lambda i: (0, i) + tuple(0 for _ in shape[2:]))


def kernel(x, edge_index, batch, W1, a1s, a1d, b1, Wp1, bp1, W2, a2s, a2d, b2,
           Wp2, bp2, W3, a3s, a3d, b3, Wl, bl):
    f32 = F32
    src = edge_index[0]
    dst = edge_index[1]
    H, D = a1s.shape

    eyeH = jnp.eye(H, dtype=f32)
    A1 = (eyeH[:, None, :] * a1s[:, :, None]).reshape(H * D, H)
    A2 = (eyeH[:, None, :] * a1d[:, :, None]).reshape(H * D, H)
    Ab = jnp.concatenate([A1, A2], axis=1)

    h, aval = pl.pallas_call(
        _k1_body,
        grid=(GRID,),
        in_specs=[_rows((BLK, 128)), _full((128, 256)), _full((256, 8))],
        out_specs=[_rows((BLK, 256)), _rows((BLK, 8))],
        out_shape=[jax.ShapeDtypeStruct((N, 256), f32),
                   jax.ShapeDtypeStruct((N, 8), f32)],
    )(x, W1, Ab)

    hsplit = h.reshape(N, 2, 128).reshape(2 * N, 128)
    aflat = aval.reshape(8 * N)
    zcomb = jnp.zeros((N, 128), f32)
    zden = jnp.zeros((N, 4), f32)

    acc, den4 = _gat_sweep_call(src, dst, hsplit, aflat, zcomb, zden)

    x1, g, dinv8 = pl.pallas_call(
        _k3_body,
        grid=(GRID,),
        in_specs=[_rows3((2, BLK, 128)), _rows3((2, BLK, 4)),
                  _full((1, 64)), _full((64, 32))],
        out_specs=[_rows((BLK, 64)), _rows((BLK, 32)), _rows((BLK, 8))],
        out_shape=[jax.ShapeDtypeStruct((N, 64), f32),
                   jax.ShapeDtypeStruct((N, 32), f32),
                   jax.ShapeDtypeStruct((N, 8), f32)],
    )(acc, den4, b1.reshape(1, 64), Wp1)

    gsplit = g.reshape(N, 2, 16).reshape(2 * N, 16)
    ztab = jnp.zeros((N, 16), f32)
    agg = _seg_sweep_call(src, dst, gsplit, ztab)

    ssoft, xp = pl.pallas_call(
        _k5_body,
        grid=(GRID,),
        in_specs=[_rows3((2, BLK, 16)), _rows((BLK, 32)), _rows((BLK, 8)),
                  _rows((BLK, 64)), _full((1, 32))],
        out_specs=[_rows((BLK, 32)), _full((32, 64))],
        out_shape=[jax.ShapeDtypeStruct((N, 32), f32),
                   jax.ShapeDtypeStruct((32, 64), f32)],
        scratch_shapes=[pltpu.VMEM((32, 64), f32)],
    )(agg, g, dinv8, x1, bp1.reshape(1, 32))

    ssplit = ssoft.reshape(N, 2, 16).reshape(2 * N, 16)
    tpart = _seg_sweep_call(src, dst, ssplit, ztab)

    out = pl.pallas_call(
        _k7_body,
        grid=(GRID,),
        in_specs=[_rows3((2, BLK, 16)), _rows((BLK, 32)), _full((32, 64)),
                  _full((64, 128)), _full((4, 32)), _full((4, 32)),
                  _full((1, 32)), _full((32, 16)), _full((1, 16)),
                  _full((32, 64)), _full((4, 16)), _full((4, 16)),
                  _full((1, 16)), _full((16, 8)), _full((1, 8))],
        out_specs=pl.BlockSpec((1, 8), lambda i: (0, 0)),
        out_shape=jax.ShapeDtypeStruct((1, 8), f32),
        scratch_shapes=[pltpu.VMEM((32, 32), f32)],
    )(tpart, ssoft, xp, W2, a2s, a2d, b2.reshape(1, 32), Wp2,
      bp2.reshape(1, 16), W3, a3s, a3d, b3.reshape(1, 16), Wl,
      bl.reshape(1, 8))
    return out


# speedup vs baseline: 22.5750x; 1.2886x over previous
import functools

import jax
import jax.numpy as jnp
from jax import lax
from jax.experimental import pallas as pl
from jax.experimental.pallas import tpu as pltpu
from jax.experimental.pallas import tpu_sc as plsc

N = 10000
E = 160000
HI = jax.lax.Precision.HIGHEST
F32 = jnp.float32

NC, NS, L = 2, 16, 16
CPR = 624
TAIL0 = NS * CPR
TAILN = N - TAIL0
EPT = E // NS
CHUNK = 80
NCHUNK = EPT // CHUNK
DLOC = 4 * N
DPAD = 40960

_MESH = plsc.VectorSubcoreMesh(
    core_axis_name="c", subcore_axis_name="s", num_cores=NC, num_subcores=NS)


def _iota16():
    return lax.iota(jnp.int32, L)


def _splat_i32(val):
    return jnp.zeros((L,), jnp.int32) + val


def _gat_sweep(src_ref, dst_ref, hsp_ref, aval_ref, zero_ref, zden_ref,
               out_ref, den_ref,
               sidx, didx, hidx, iaa, iab, vab, vdb,
               hbuf, comb, denpay, acc_sh, den_sh, sem):
    c = lax.axis_index("c")
    s = lax.axis_index("s")
    r0 = s * CPR
    pltpu.sync_copy(zero_ref.at[pl.ds(r0, CPR)], acc_sh.at[pl.ds(r0, CPR)])
    pltpu.sync_copy(zden_ref.at[pl.ds(r0, CPR)], den_sh.at[pl.ds(r0, CPR)])

    @pl.when(s == 0)
    def _():
        pltpu.sync_copy(zero_ref.at[pl.ds(TAIL0, TAILN)],
                        acc_sh.at[pl.ds(TAIL0, TAILN)])
        pltpu.sync_copy(zden_ref.at[pl.ds(TAIL0, TAILN)],
                        den_sh.at[pl.ds(TAIL0, TAILN)])

    plsc.subcore_barrier()
    ones = jnp.zeros((L,), F32) + 1.0
    zeros = jnp.zeros((L,), F32)
    for g in range(CHUNK // L):
        rows = _iota16() + g * L
        plsc.store_scatter(denpay, [rows, _splat_i32(2)], ones)
        plsc.store_scatter(denpay, [rows, _splat_i32(3)], zeros)

    def body(k, _):
        base = s * EPT + k * CHUNK
        d_s = pltpu.async_copy(src_ref.at[pl.ds(base, CHUNK)], sidx, sem)
        d_d = pltpu.async_copy(dst_ref.at[pl.ds(base, CHUNK)], didx, sem)
        d_s.wait()
        d_d.wait()
        for g in range(CHUNK // L):
            sv = sidx[pl.ds(g * L, L)]
            dv = didx[pl.ds(g * L, L)]
            hidx[pl.ds(g * L, L)] = sv * 2 + c
            iaa[pl.ds(g * L, L)] = sv * 4 + c
            iab[pl.ds(g * L, L)] = dv * 4 + 2 + c
        d_a = pltpu.async_copy(aval_ref.at[iaa], vab, sem)
        d_b = pltpu.async_copy(aval_ref.at[iab], vdb, sem)
        d_h = pltpu.async_copy(hsp_ref.at[hidx], hbuf, sem)
        d_a.wait()
        d_b.wait()
        half = _iota16() >> 1
        par = _iota16() & 1
        for g in range(2 * CHUNK // L):
            rows = half + g * (L // 2)
            lg = (plsc.load_gather(vab, [rows, par])
                  + plsc.load_gather(vdb, [rows, par]))
            ex = jnp.exp(jnp.maximum(lg, 0.2 * lg))
            plsc.store_scatter(vab, [rows, par], ex)
            plsc.store_scatter(denpay, [rows, par], ex)
        d_h.wait()
        for e in range(CHUNK):
            sp0 = plsc.load_gather(vab, [_splat_i32(e), _splat_i32(0)])
            sp1 = plsc.load_gather(vab, [_splat_i32(e), _splat_i32(1)])
            for q in range(4):
                comb[e, pl.ds(q * L, L)] = hbuf[e, pl.ds(q * L, L)] * sp0
            for q in range(4, 8):
                comb[e, pl.ds(q * L, L)] = hbuf[e, pl.ds(q * L, L)] * sp1
        d_c = pltpu.async_copy(comb, acc_sh.at[didx], sem, add=True)
        d_p = pltpu.async_copy(denpay, den_sh.at[didx], sem, add=True)
        d_c.wait()
        d_p.wait()
        return ()

    lax.fori_loop(0, NCHUNK, body, ())
    plsc.subcore_barrier()
    pltpu.sync_copy(acc_sh.at[pl.ds(r0, CPR)],
                    out_ref.at[c].at[pl.ds(r0, CPR)])
    pltpu.sync_copy(den_sh.at[pl.ds(r0, CPR)],
                    den_ref.at[c].at[pl.ds(r0, CPR)])

    @pl.when(s == 0)
    def _():
        pltpu.sync_copy(acc_sh.at[pl.ds(TAIL0, TAILN)],
                        out_ref.at[c].at[pl.ds(TAIL0, TAILN)])
        pltpu.sync_copy(den_sh.at[pl.ds(TAIL0, TAILN)],
                        den_ref.at[c].at[pl.ds(TAIL0, TAILN)])


_gat_sweep_call = functools.partial(
    pl.kernel,
    out_type=(jax.ShapeDtypeStruct((NC, N, 128), F32),
              jax.ShapeDtypeStruct((NC, N, 4), F32)),
    mesh=_MESH,
    scratch_types=[
        pltpu.VMEM((CHUNK,), jnp.int32),
        pltpu.VMEM((CHUNK,), jnp.int32),
        pltpu.VMEM((CHUNK,), jnp.int32),
        pltpu.VMEM((CHUNK,), jnp.int32),
        pltpu.VMEM((CHUNK,), jnp.int32),
        pltpu.VMEM((CHUNK, 2), F32),
        pltpu.VMEM((CHUNK, 2), F32),
        pltpu.VMEM((CHUNK, 128), F32),
        pltpu.VMEM((CHUNK, 128), F32),
        pltpu.VMEM((CHUNK, 4), F32),
        pltpu.VMEM_SHARED((N, 128), F32),
        pltpu.VMEM_SHARED((N, 4), F32),
        pltpu.SemaphoreType.DMA,
    ],
    compiler_params=pltpu.CompilerParams(use_tc_tiling_on_sc=False,
                                        needs_layout_passes=False),
)(_gat_sweep)


def _seg_sweep(src_ref, dst_ref, tab_ref, zero_ref, out_ref,
               sidx, didx, hidx, buf, acc_sh, sem):
    c = lax.axis_index("c")
    s = lax.axis_index("s")
    r0 = s * CPR
    pltpu.sync_copy(zero_ref.at[pl.ds(r0, CPR)], acc_sh.at[pl.ds(r0, CPR)])

    @pl.when(s == 0)
    def _():
        pltpu.sync_copy(zero_ref.at[pl.ds(TAIL0, TAILN)],
                        acc_sh.at[pl.ds(TAIL0, TAILN)])

    plsc.subcore_barrier()

    def body(k, _):
        base = s * EPT + k * CHUNK
        d_s = pltpu.async_copy(src_ref.at[pl.ds(base, CHUNK)], sidx, sem)
        d_d = pltpu.async_copy(dst_ref.at[pl.ds(base, CHUNK)], didx, sem)
        d_s.wait()
        d_d.wait()
        for g in range(CHUNK // L):
            v = sidx[pl.ds(g * L, L)]
            hidx[pl.ds(g * L, L)] = v * 2 + c
        pltpu.async_copy(tab_ref.at[hidx], buf, sem).wait()
        pltpu.sync_copy(buf, acc_sh.at[didx], add=True)
        return ()

    lax.fori_loop(0, NCHUNK, body, (), unroll=False)
    plsc.subcore_barrier()
    pltpu.sync_copy(acc_sh.at[pl.ds(r0, CPR)],
                    out_ref.at[c].at[pl.ds(r0, CPR)])

    @pl.when(s == 0)
    def _():
        pltpu.sync_copy(acc_sh.at[pl.ds(TAIL0, TAILN)],
                        out_ref.at[c].at[pl.ds(TAIL0, TAILN)])


_seg_sweep_call = functools.partial(
    pl.kernel,
    out_type=jax.ShapeDtypeStruct((NC, N, 16), F32),
    mesh=_MESH,
    scratch_types=[
        pltpu.VMEM((CHUNK,), jnp.int32),
        pltpu.VMEM((CHUNK,), jnp.int32),
        pltpu.VMEM((CHUNK,), jnp.int32),
        pltpu.VMEM((CHUNK, 16), F32),
        pltpu.VMEM_SHARED((N, 16), F32),
        pltpu.SemaphoreType.DMA,
    ],
    compiler_params=pltpu.CompilerParams(use_tc_tiling_on_sc=False,
                                        needs_layout_passes=False),
)(_seg_sweep)


BLK = 1000
GRID = N // BLK


def _k1_body(x_ref, w1_ref, ab_ref, h_ref, aval_ref):
    h = jnp.dot(x_ref[...], w1_ref[...], precision=HI)
    h_ref[...] = h
    aval_ref[...] = jnp.dot(h, ab_ref[...], precision=HI)


def _k2b_body(dp_ref, den_ref):
    den_ref[...] = jnp.sum(dp_ref[...], axis=1)


def _k3_body(acc_ref, den_ref, b1_ref, wp1_ref, x1_ref, g_ref, dinv_ref):
    a0 = acc_ref[0]
    a1 = acc_ref[1]
    d0 = den_ref[0]
    d1 = den_ref[1]
    eps = 1e-16
    x1 = 0.25 * (a0[:, 0:64] / (d0[:, 0:1] + eps)
                 + a0[:, 64:128] / (d0[:, 1:2] + eps)
                 + a1[:, 0:64] / (d1[:, 0:1] + eps)
                 + a1[:, 64:128] / (d1[:, 1:2] + eps)) + b1_ref[...]
    deg = d0[:, 2:3] + 1.0
    dinv = lax.rsqrt(deg)
    h1 = jnp.dot(x1, wp1_ref[...], precision=HI)
    x1_ref[...] = x1
    g_ref[...] = dinv * h1
    dinv_ref[...] = jnp.broadcast_to(dinv, (BLK, 8))


def _k5_body(agg_ref, g_ref, dinv_ref, x1_ref, bp1_ref,
             ssoft_ref, xp_ref, xp_acc):
    i = pl.program_id(0)
    aggcat = jnp.concatenate([agg_ref[0], agg_ref[1]], axis=1)
    dinv = dinv_ref[:, 0:1]
    s1 = dinv * (aggcat + g_ref[...]) + bp1_ref[...]
    m = jnp.max(s1, axis=1, keepdims=True)
    ee = jnp.exp(s1 - m)
    ssoft = ee / jnp.sum(ee, axis=1, keepdims=True)
    ssoft_ref[...] = ssoft

    @pl.when(i == 0)
    def _():
        xp_acc[...] = jnp.zeros_like(xp_acc)

    xp_acc[...] += lax.dot_general(ssoft, x1_ref[...],
                                   (((0,), (0,)), ((), ())), precision=HI)

    @pl.when(i == GRID - 1)
    def _():
        xp_ref[...] = xp_acc[...]


def _gat_dense_tc(x, adj, W, a_s, a_d, b, H, D):
    h = jnp.dot(x, W, precision=HI)
    ladj = jnp.log(adj + 1e-9)
    acc = None
    for hh in range(H):
        hv = h[:, hh * D:(hh + 1) * D]
        als_col = lax.dot_general(hv, a_s[hh:hh + 1, :],
                                  (((1,), (1,)), ((), ())), precision=HI)
        ald_row = lax.dot_general(a_d[hh:hh + 1, :], hv,
                                  (((1,), (1,)), ((), ())), precision=HI)
        lg = als_col + ald_row
        Lm = jnp.maximum(lg, 0.2 * lg) + ladj
        m = jnp.max(Lm, axis=1, keepdims=True)
        ee = jnp.exp(Lm - m)
        alpha = ee / jnp.sum(ee, axis=1, keepdims=True)
        outh = jnp.dot(alpha, hv, precision=HI)
        acc = outh if acc is None else acc + outh
    return acc / H + b


def _k7_body(t_ref, ssoft_ref, xp_ref, w2_ref, a2s_ref, a2d_ref, b2_ref,
             wp2_ref, bp2_ref, w3_ref, a3s_ref, a3d_ref, b3_ref,
             wl_ref, bl_ref, out_ref, adj_acc):
    i = pl.program_id(0)

    @pl.when(i == 0)
    def _():
        adj_acc[...] = jnp.zeros_like(adj_acc)

    tcat = jnp.concatenate([t_ref[0], t_ref[1]], axis=1)
    adj_acc[...] += lax.dot_general(tcat, ssoft_ref[...],
                                    (((0,), (0,)), ((), ())), precision=HI)

    @pl.when(i == GRID - 1)
    def _():
        adjp = adj_acc[...]
        x2 = xp_ref[...]
        x3 = _gat_dense_tc(x2, adjp, w2_ref[...], a2s_ref[...], a2d_ref[...],
                           b2_ref[...], 4, 32)
        n = 32
        ii = lax.broadcasted_iota(jnp.int32, (n, n), 0)
        jj = lax.broadcasted_iota(jnp.int32, (n, n), 1)
        A = adjp + jnp.where(ii == jj, 1.0, 0.0)
        deg = jnp.sum(A, axis=1, keepdims=True)
        dinv = lax.rsqrt(deg)
        y = dinv * jnp.dot(x3, wp2_ref[...], precision=HI)
        s2 = dinv * jnp.dot(A, y, precision=HI) + bp2_ref[...]
        m = jnp.max(s2, axis=1, keepdims=True)
        ee = jnp.exp(s2 - m)
        s = ee / jnp.sum(ee, axis=1, keepdims=True)
        x4 = lax.dot_general(s, x3, (((0,), (0,)), ((), ())), precision=HI)
        adj3 = lax.dot_general(s, jnp.dot(adjp, s, precision=HI),
                               (((0,), (0,)), ((), ())), precision=HI)
        x5 = _gat_dense_tc(x4, adj3, w3_ref[...], a3s_ref[...], a3d_ref[...],
                           b3_ref[...], 4, 16)
        x6 = jnp.dot(x5, wl_ref[...], precision=HI) + bl_ref[...]
        out_ref[...] = jnp.mean(x6, axis=0, keepdims=True)


def _full(shape):
    return pl.BlockSpec(shape, lambda i: tuple(0 for _ in shape))


def _rows(shape):
    return pl.BlockSpec(shape, lambda i: (i,) + tuple(0 for _ in shape[1:]))


def _rows3(shape):
    return pl.BlockSpec(shape, lambda i: (0, i) + tuple(0 for _ in shape[2:]))


def kernel(x, edge_index, batch, W1, a1s, a1d, b1, Wp1, bp1, W2, a2s, a2d, b2,
           Wp2, bp2, W3, a3s, a3d, b3, Wl, bl):
    f32 = F32
    src = edge_index[0]
    dst = edge_index[1]
    H, D = a1s.shape

    eyeH = jnp.eye(H, dtype=f32)
    A1 = (eyeH[:, None, :] * a1s[:, :, None]).reshape(H * D, H)
    A2 = (eyeH[:, None, :] * a1d[:, :, None]).reshape(H * D, H)
    Ab = jnp.concatenate([A1, A2], axis=1)

    h, aval = pl.pallas_call(
        _k1_body,
        grid=(GRID,),
        in_specs=[_rows((BLK, 128)), _full((128, 256)), _full((256, 8))],
        out_specs=[_rows((BLK, 256)), _rows((BLK, 8))],
        out_shape=[jax.ShapeDtypeStruct((N, 256), f32),
                   jax.ShapeDtypeStruct((N, 8), f32)],
    )(x, W1, Ab)

    hsplit = h.reshape(N, 2, 128).reshape(2 * N, 128)
    avalp = aval.reshape(4 * N, 2)
    zcomb = jnp.zeros((N, 128), f32)
    zden = jnp.zeros((N, 4), f32)

    acc, den4 = _gat_sweep_call(src, dst, hsplit, avalp, zcomb, zden)

    x1, g, dinv8 = pl.pallas_call(
        _k3_body,
        grid=(GRID,),
        in_specs=[_rows3((2, BLK, 128)), _rows3((2, BLK, 4)),
                  _full((1, 64)), _full((64, 32))],
        out_specs=[_rows((BLK, 64)), _rows((BLK, 32)), _rows((BLK, 8))],
        out_shape=[jax.ShapeDtypeStruct((N, 64), f32),
                   jax.ShapeDtypeStruct((N, 32), f32),
                   jax.ShapeDtypeStruct((N, 8), f32)],
    )(acc, den4, b1.reshape(1, 64), Wp1)

    gsplit = g.reshape(N, 2, 16).reshape(2 * N, 16)
    ztab = jnp.zeros((N, 16), f32)
    agg = _seg_sweep_call(src, dst, gsplit, ztab)

    ssoft, xp = pl.pallas_call(
        _k5_body,
        grid=(GRID,),
        in_specs=[_rows3((2, BLK, 16)), _rows((BLK, 32)), _rows((BLK, 8)),
                  _rows((BLK, 64)), _full((1, 32))],
        out_specs=[_rows((BLK, 32)), _full((32, 64))],
        out_shape=[jax.ShapeDtypeStruct((N, 32), f32),
                   jax.ShapeDtypeStruct((32, 64), f32)],
        scratch_shapes=[pltpu.VMEM((32, 64), f32)],
    )(agg, g, dinv8, x1, bp1.reshape(1, 32))

    ssplit = ssoft.reshape(N, 2, 16).reshape(2 * N, 16)
    tpart = _seg_sweep_call(src, dst, ssplit, ztab)

    out = pl.pallas_call(
        _k7_body,
        grid=(GRID,),
        in_specs=[_rows3((2, BLK, 16)), _rows((BLK, 32)), _full((32, 64)),
                  _full((64, 128)), _full((4, 32)), _full((4, 32)),
                  _full((1, 32)), _full((32, 16)), _full((1, 16)),
                  _full((32, 64)), _full((4, 16)), _full((4, 16)),
                  _full((1, 16)), _full((16, 8)), _full((1, 8))],
        out_specs=pl.BlockSpec((1, 8), lambda i: (0, 0)),
        out_shape=jax.ShapeDtypeStruct((1, 8), f32),
        scratch_shapes=[pltpu.VMEM((32, 32), f32)],
    )(tpart, ssoft, xp, W2, a2s, a2d, b2.reshape(1, 32), Wp2,
      bp2.reshape(1, 16), W3, a3s, a3d, b3.reshape(1, 16), Wl,
      bl.reshape(1, 8))
    return out


# speedup vs baseline: 25.3747x; 1.1240x over previous
import functools

import jax
import jax.numpy as jnp
from jax import lax
from jax.experimental import pallas as pl
from jax.experimental.pallas import tpu as pltpu
from jax.experimental.pallas import tpu_sc as plsc

N = 10000
E = 160000
HI = jax.lax.Precision.HIGHEST
F32 = jnp.float32

NC, NS, L = 2, 16, 16
CPR = 624
TAIL0 = NS * CPR
TAILN = N - TAIL0
EPT = E // NS
CHUNK = 80
NCHUNK = EPT // CHUNK
DLOC = 4 * N
DPAD = 40960

_MESH = plsc.VectorSubcoreMesh(
    core_axis_name="c", subcore_axis_name="s", num_cores=NC, num_subcores=NS)


def _iota16():
    return lax.iota(jnp.int32, L)


def _splat_i32(val):
    return jnp.zeros((L,), jnp.int32) + val


NBUF = 2
PAIRS = NCHUNK // NBUF


def _gat_sweep(src_ref, dst_ref, hsp_ref, aval_ref, zero_ref, zden_ref,
               out_ref, den_ref, *scr):
    bufs = [scr[i * 11:(i + 1) * 11] for i in range(NBUF)]
    acc_sh, den_sh = scr[22], scr[23]
    sems = [scr[24 + i * 3:24 + (i + 1) * 3] for i in range(NBUF)]
    c = lax.axis_index("c")
    s = lax.axis_index("s")
    r0 = s * CPR
    pltpu.sync_copy(zero_ref.at[pl.ds(r0, CPR)], acc_sh.at[pl.ds(r0, CPR)])
    pltpu.sync_copy(zden_ref.at[pl.ds(r0, CPR)], den_sh.at[pl.ds(r0, CPR)])

    @pl.when(s == 0)
    def _():
        pltpu.sync_copy(zero_ref.at[pl.ds(TAIL0, TAILN)],
                        acc_sh.at[pl.ds(TAIL0, TAILN)])
        pltpu.sync_copy(zden_ref.at[pl.ds(TAIL0, TAILN)],
                        den_sh.at[pl.ds(TAIL0, TAILN)])

    plsc.subcore_barrier()
    ones = jnp.zeros((L,), F32) + 1.0
    zeros = jnp.zeros((L,), F32)
    for b in range(NBUF):
        denpay = bufs[b][9]
        for g in range(CHUNK // L):
            rows = _iota16() + g * L
            plsc.store_scatter(denpay, [rows, _splat_i32(2)], ones)
            plsc.store_scatter(denpay, [rows, _splat_i32(3)], zeros)

    def s1(b, k):
        sidx, didx, hidx, iaa, iab, vab, vdb, hbuf, comb, denpay, didx2 = bufs[b]
        semi, semg, semc = sems[b]
        base = s * EPT + k * CHUNK
        d_s = pltpu.async_copy(src_ref.at[pl.ds(base, CHUNK)], sidx, semi)
        d_d = pltpu.async_copy(dst_ref.at[pl.ds(base, CHUNK)], didx, semi)
        d_s.wait()
        d_d.wait()
        for g in range(CHUNK // L):
            sv = sidx[pl.ds(g * L, L)]
            dv = didx[pl.ds(g * L, L)]
            hidx[pl.ds(g * L, L)] = sv * 2 + c
            iaa[pl.ds(g * L, L)] = sv * 4 + c
            iab[pl.ds(g * L, L)] = dv * 4 + 2 + c
        pltpu.async_copy(aval_ref.at[iaa], vab, semg)
        pltpu.async_copy(aval_ref.at[iab], vdb, semg)
        pltpu.async_copy(hsp_ref.at[hidx], hbuf, semg)

    def s2(b, drain):
        sidx, didx, hidx, iaa, iab, vab, vdb, hbuf, comb, denpay, didx2 = bufs[b]
        semi, semg, semc = sems[b]
        pltpu.make_async_copy(aval_ref.at[iaa], vab, semg).wait()
        pltpu.make_async_copy(aval_ref.at[iab], vdb, semg).wait()

        def do_drain():
            pltpu.make_async_copy(comb, acc_sh.at[didx2], semc).wait()
            pltpu.make_async_copy(denpay, den_sh.at[didx2], semc).wait()

        if drain is None:
            do_drain()
        else:
            pl.when(drain)(do_drain)
        half = _iota16() >> 1
        par = _iota16() & 1
        for g in range(2 * CHUNK // L):
            rows = half + g * (L // 2)
            lg = (plsc.load_gather(vab, [rows, par])
                  + plsc.load_gather(vdb, [rows, par]))
            ex = jnp.exp(jnp.maximum(lg, 0.2 * lg))
            plsc.store_scatter(vab, [rows, par], ex)
            plsc.store_scatter(denpay, [rows, par], ex)
        pltpu.make_async_copy(hsp_ref.at[hidx], hbuf, semg).wait()
        for e in range(CHUNK):
            sp0 = plsc.load_gather(vab, [_splat_i32(e), _splat_i32(0)])
            sp1 = plsc.load_gather(vab, [_splat_i32(e), _splat_i32(1)])
            for q in range(4):
                comb[e, pl.ds(q * L, L)] = hbuf[e, pl.ds(q * L, L)] * sp0
            for q in range(4, 8):
                comb[e, pl.ds(q * L, L)] = hbuf[e, pl.ds(q * L, L)] * sp1
        for g in range(CHUNK // L):
            didx2[pl.ds(g * L, L)] = didx[pl.ds(g * L, L)]
        pltpu.async_copy(comb, acc_sh.at[didx2], semc, add=True)
        pltpu.async_copy(denpay, den_sh.at[didx2], semc, add=True)

    s1(0, 0)
    s1(1, 1)

    def body(t, _):
        k0 = 2 * t
        s2(0, drain=t > 0)
        s1(0, k0 + 2)

        s2(1, drain=t > 0)

        @pl.when(k0 + 3 < NCHUNK)
        def _():
            s1(1, k0 + 3)

        return ()

    lax.fori_loop(0, PAIRS, body, ())
    s2(0, drain=None)
    _, didxb, _, _, _, _, _, _, combb, denpayb, didx2b = bufs[1]
    pltpu.make_async_copy(combb, acc_sh.at[didx2b], sems[1][2]).wait()
    pltpu.make_async_copy(denpayb, den_sh.at[didx2b], sems[1][2]).wait()
    _, _, _, _, _, _, _, _, comba, denpaya, didx2a = bufs[0]
    pltpu.make_async_copy(comba, acc_sh.at[didx2a], sems[0][2]).wait()
    pltpu.make_async_copy(denpaya, den_sh.at[didx2a], sems[0][2]).wait()
    plsc.subcore_barrier()
    pltpu.sync_copy(acc_sh.at[pl.ds(r0, CPR)],
                    out_ref.at[c].at[pl.ds(r0, CPR)])
    pltpu.sync_copy(den_sh.at[pl.ds(r0, CPR)],
                    den_ref.at[c].at[pl.ds(r0, CPR)])

    @pl.when(s == 0)
    def _():
        pltpu.sync_copy(acc_sh.at[pl.ds(TAIL0, TAILN)],
                        out_ref.at[c].at[pl.ds(TAIL0, TAILN)])
        pltpu.sync_copy(den_sh.at[pl.ds(TAIL0, TAILN)],
                        den_ref.at[c].at[pl.ds(TAIL0, TAILN)])


_gat_sweep_call = functools.partial(
    pl.kernel,
    out_type=(jax.ShapeDtypeStruct((NC, N, 128), F32),
              jax.ShapeDtypeStruct((NC, N, 4), F32)),
    mesh=_MESH,
    scratch_types=(
        [pltpu.VMEM((CHUNK,), jnp.int32),
         pltpu.VMEM((CHUNK,), jnp.int32),
         pltpu.VMEM((CHUNK,), jnp.int32),
         pltpu.VMEM((CHUNK,), jnp.int32),
         pltpu.VMEM((CHUNK,), jnp.int32),
         pltpu.VMEM((CHUNK, 2), F32),
         pltpu.VMEM((CHUNK, 2), F32),
         pltpu.VMEM((CHUNK, 128), F32),
         pltpu.VMEM((CHUNK, 128), F32),
         pltpu.VMEM((CHUNK, 4), F32),
         pltpu.VMEM((CHUNK,), jnp.int32),
         ] * NBUF
        + [pltpu.VMEM_SHARED((N, 128), F32),
           pltpu.VMEM_SHARED((N, 4), F32)]
        + [pltpu.SemaphoreType.DMA] * (3 * NBUF)
    ),
    compiler_params=pltpu.CompilerParams(use_tc_tiling_on_sc=False,
                                        needs_layout_passes=False),
)(_gat_sweep)


def _seg_sweep(src_ref, dst_ref, tab_ref, zero_ref, out_ref,
               sidx, didx, hidx, buf, acc_sh, sem):
    c = lax.axis_index("c")
    s = lax.axis_index("s")
    r0 = s * CPR
    pltpu.sync_copy(zero_ref.at[pl.ds(r0, CPR)], acc_sh.at[pl.ds(r0, CPR)])

    @pl.when(s == 0)
    def _():
        pltpu.sync_copy(zero_ref.at[pl.ds(TAIL0, TAILN)],
                        acc_sh.at[pl.ds(TAIL0, TAILN)])

    plsc.subcore_barrier()

    def body(k, _):
        base = s * EPT + k * CHUNK
        d_s = pltpu.async_copy(src_ref.at[pl.ds(base, CHUNK)], sidx, sem)
        d_d = pltpu.async_copy(dst_ref.at[pl.ds(base, CHUNK)], didx, sem)
        d_s.wait()
        d_d.wait()
        for g in range(CHUNK // L):
            v = sidx[pl.ds(g * L, L)]
            hidx[pl.ds(g * L, L)] = v * 2 + c
        pltpu.async_copy(tab_ref.at[hidx], buf, sem).wait()
        pltpu.sync_copy(buf, acc_sh.at[didx], add=True)
        return ()

    lax.fori_loop(0, NCHUNK, body, (), unroll=False)
    plsc.subcore_barrier()
    pltpu.sync_copy(acc_sh.at[pl.ds(r0, CPR)],
                    out_ref.at[c].at[pl.ds(r0, CPR)])

    @pl.when(s == 0)
    def _():
        pltpu.sync_copy(acc_sh.at[pl.ds(TAIL0, TAILN)],
                        out_ref.at[c].at[pl.ds(TAIL0, TAILN)])


_seg_sweep_call = functools.partial(
    pl.kernel,
    out_type=jax.ShapeDtypeStruct((NC, N, 16), F32),
    mesh=_MESH,
    scratch_types=[
        pltpu.VMEM((CHUNK,), jnp.int32),
        pltpu.VMEM((CHUNK,), jnp.int32),
        pltpu.VMEM((CHUNK,), jnp.int32),
        pltpu.VMEM((CHUNK, 16), F32),
        pltpu.VMEM_SHARED((N, 16), F32),
        pltpu.SemaphoreType.DMA,
    ],
    compiler_params=pltpu.CompilerParams(use_tc_tiling_on_sc=False,
                                        needs_layout_passes=False),
)(_seg_sweep)


BLK = 1000
GRID = N // BLK


def _k1_body(x_ref, w1_ref, ab_ref, h_ref, aval_ref):
    h = jnp.dot(x_ref[...], w1_ref[...], precision=HI)
    h_ref[...] = h
    aval_ref[...] = jnp.dot(h, ab_ref[...], precision=HI)


def _k2b_body(dp_ref, den_ref):
    den_ref[...] = jnp.sum(dp_ref[...], axis=1)


def _k3_body(acc_ref, den_ref, b1_ref, wp1_ref, x1_ref, g_ref, dinv_ref):
    a0 = acc_ref[0]
    a1 = acc_ref[1]
    d0 = den_ref[0]
    d1 = den_ref[1]
    eps = 1e-16
    x1 = 0.25 * (a0[:, 0:64] / (d0[:, 0:1] + eps)
                 + a0[:, 64:128] / (d0[:, 1:2] + eps)
                 + a1[:, 0:64] / (d1[:, 0:1] + eps)
                 + a1[:, 64:128] / (d1[:, 1:2] + eps)) + b1_ref[...]
    deg = d0[:, 2:3] + 1.0
    dinv = lax.rsqrt(deg)
    h1 = jnp.dot(x1, wp1_ref[...], precision=HI)
    x1_ref[...] = x1
    g_ref[...] = dinv * h1
    dinv_ref[...] = jnp.broadcast_to(dinv, (BLK, 8))


def _k5_body(agg_ref, g_ref, dinv_ref, x1_ref, bp1_ref,
             ssoft_ref, xp_ref, xp_acc):
    i = pl.program_id(0)
    aggcat = jnp.concatenate([agg_ref[0], agg_ref[1]], axis=1)
    dinv = dinv_ref[:, 0:1]
    s1 = dinv * (aggcat + g_ref[...]) + bp1_ref[...]
    m = jnp.max(s1, axis=1, keepdims=True)
    ee = jnp.exp(s1 - m)
    ssoft = ee / jnp.sum(ee, axis=1, keepdims=True)
    ssoft_ref[...] = ssoft

    @pl.when(i == 0)
    def _():
        xp_acc[...] = jnp.zeros_like(xp_acc)

    xp_acc[...] += lax.dot_general(ssoft, x1_ref[...],
                                   (((0,), (0,)), ((), ())), precision=HI)

    @pl.when(i == GRID - 1)
    def _():
        xp_ref[...] = xp_acc[...]


def _gat_dense_tc(x, adj, W, a_s, a_d, b, H, D):
    h = jnp.dot(x, W, precision=HI)
    ladj = jnp.log(adj + 1e-9)
    acc = None
    for hh in range(H):
        hv = h[:, hh * D:(hh + 1) * D]
        als_col = lax.dot_general(hv, a_s[hh:hh + 1, :],
                                  (((1,), (1,)), ((), ())), precision=HI)
        ald_row = lax.dot_general(a_d[hh:hh + 1, :], hv,
                                  (((1,), (1,)), ((), ())), precision=HI)
        lg = als_col + ald_row
        Lm = jnp.maximum(lg, 0.2 * lg) + ladj
        m = jnp.max(Lm, axis=1, keepdims=True)
        ee = jnp.exp(Lm - m)
        alpha = ee / jnp.sum(ee, axis=1, keepdims=True)
        outh = jnp.dot(alpha, hv, precision=HI)
        acc = outh if acc is None else acc + outh
    return acc / H + b


def _k7_body(t_ref, ssoft_ref, xp_ref, w2_ref, a2s_ref, a2d_ref, b2_ref,
             wp2_ref, bp2_ref, w3_ref, a3s_ref, a3d_ref, b3_ref,
             wl_ref, bl_ref, out_ref, adj_acc):
    i = pl.program_id(0)

    @pl.when(i == 0)
    def _():
        adj_acc[...] = jnp.zeros_like(adj_acc)

    tcat = jnp.concatenate([t_ref[0], t_ref[1]], axis=1)
    adj_acc[...] += lax.dot_general(tcat, ssoft_ref[...],
                                    (((0,), (0,)), ((), ())), precision=HI)

    @pl.when(i == GRID - 1)
    def _():
        adjp = adj_acc[...]
        x2 = xp_ref[...]
        x3 = _gat_dense_tc(x2, adjp, w2_ref[...], a2s_ref[...], a2d_ref[...],
                           b2_ref[...], 4, 32)
        n = 32
        ii = lax.broadcasted_iota(jnp.int32, (n, n), 0)
        jj = lax.broadcasted_iota(jnp.int32, (n, n), 1)
        A = adjp + jnp.where(ii == jj, 1.0, 0.0)
        deg = jnp.sum(A, axis=1, keepdims=True)
        dinv = lax.rsqrt(deg)
        y = dinv * jnp.dot(x3, wp2_ref[...], precision=HI)
        s2 = dinv * jnp.dot(A, y, precision=HI) + bp2_ref[...]
        m = jnp.max(s2, axis=1, keepdims=True)
        ee = jnp.exp(s2 - m)
        s = ee / jnp.sum(ee, axis=1, keepdims=True)
        x4 = lax.dot_general(s, x3, (((0,), (0,)), ((), ())), precision=HI)
        adj3 = lax.dot_general(s, jnp.dot(adjp, s, precision=HI),
                               (((0,), (0,)), ((), ())), precision=HI)
        x5 = _gat_dense_tc(x4, adj3, w3_ref[...], a3s_ref[...], a3d_ref[...],
                           b3_ref[...], 4, 16)
        x6 = jnp.dot(x5, wl_ref[...], precision=HI) + bl_ref[...]
        out_ref[...] = jnp.mean(x6, axis=0, keepdims=True)


def _full(shape):
    return pl.BlockSpec(shape, lambda i: tuple(0 for _ in shape))


def _rows(shape):
    return pl.BlockSpec(shape, lambda i: (i,) + tuple(0 for _ in shape[1:]))


def _rows3(shape):
    return pl.BlockSpec(shape, lambda i: (0, i) + tuple(0 for _ in shape[2:]))


def kernel(x, edge_index, batch, W1, a1s, a1d, b1, Wp1, bp1, W2, a2s, a2d, b2,
           Wp2, bp2, W3, a3s, a3d, b3, Wl, bl):
    f32 = F32
    src = edge_index[0]
    dst = edge_index[1]
    H, D = a1s.shape

    eyeH = jnp.eye(H, dtype=f32)
    A1 = (eyeH[:, None, :] * a1s[:, :, None]).reshape(H * D, H)
    A2 = (eyeH[:, None, :] * a1d[:, :, None]).reshape(H * D, H)
    Ab = jnp.concatenate([A1, A2], axis=1)

    h, aval = pl.pallas_call(
        _k1_body,
        grid=(GRID,),
        in_specs=[_rows((BLK, 128)), _full((128, 256)), _full((256, 8))],
        out_specs=[_rows((BLK, 256)), _rows((BLK, 8))],
        out_shape=[jax.ShapeDtypeStruct((N, 256), f32),
                   jax.ShapeDtypeStruct((N, 8), f32)],
    )(x, W1, Ab)

    hsplit = h.reshape(N, 2, 128).reshape(2 * N, 128)
    avalp = aval.reshape(4 * N, 2)
    zcomb = jnp.zeros((N, 128), f32)
    zden = jnp.zeros((N, 4), f32)

    acc, den4 = _gat_sweep_call(src, dst, hsplit, avalp, zcomb, zden)

    x1, g, dinv8 = pl.pallas_call(
        _k3_body,
        grid=(GRID,),
        in_specs=[_rows3((2, BLK, 128)), _rows3((2, BLK, 4)),
                  _full((1, 64)), _full((64, 32))],
        out_specs=[_rows((BLK, 64)), _rows((BLK, 32)), _rows((BLK, 8))],
        out_shape=[jax.ShapeDtypeStruct((N, 64), f32),
                   jax.ShapeDtypeStruct((N, 32), f32),
                   jax.ShapeDtypeStruct((N, 8), f32)],
    )(acc, den4, b1.reshape(1, 64), Wp1)

    gsplit = g.reshape(N, 2, 16).reshape(2 * N, 16)
    ztab = jnp.zeros((N, 16), f32)
    agg = _seg_sweep_call(src, dst, gsplit, ztab)

    ssoft, xp = pl.pallas_call(
        _k5_body,
        grid=(GRID,),
        in_specs=[_rows3((2, BLK, 16)), _rows((BLK, 32)), _rows((BLK, 8)),
                  _rows((BLK, 64)), _full((1, 32))],
        out_specs=[_rows((BLK, 32)), _full((32, 64))],
        out_shape=[jax.ShapeDtypeStruct((N, 32), f32),
                   jax.ShapeDtypeStruct((32, 64), f32)],
        scratch_shapes=[pltpu.VMEM((32, 64), f32)],
    )(agg, g, dinv8, x1, bp1.reshape(1, 32))

    ssplit = ssoft.reshape(N, 2, 16).reshape(2 * N, 16)
    tpart = _seg_sweep_call(src, dst, ssplit, ztab)

    out = pl.pallas_call(
        _k7_body,
        grid=(GRID,),
        in_specs=[_rows3((2, BLK, 16)), _rows((BLK, 32)), _full((32, 64)),
                  _full((64, 128)), _full((4, 32)), _full((4, 32)),
                  _full((1, 32)), _full((32, 16)), _full((1, 16)),
                  _full((32, 64)), _full((4, 16)), _full((4, 16)),
                  _full((1, 16)), _full((16, 8)), _full((1, 8))],
        out_specs=pl.BlockSpec((1, 8), lambda i: (0, 0)),
        out_shape=jax.ShapeDtypeStruct((1, 8), f32),
        scratch_shapes=[pltpu.VMEM((32, 32), f32)],
    )(tpart, ssoft, xp, W2, a2s, a2d, b2.reshape(1, 32), Wp2,
      bp2.reshape(1, 16), W3, a3s, a3d, b3.reshape(1, 16), Wl,
      bl.reshape(1, 8))
    return out
